# Initial kernel scaffold; baseline (speedup 1.0000x reference)
import jax
import jax.numpy as jnp
from jax.experimental import pallas as pl


def kernel(x, weight, bias):
    raise NotImplementedError("write your pallas kernel here")


# speedup vs baseline: 2.3712x; 2.3712x over previous
import functools

import jax
import jax.numpy as jnp
from jax.experimental import pallas as pl
from jax.experimental.pallas import tpu as pltpu


def _matmul_body(x_ref, w_ref, b_ref, o_ref, xm_ref, *, topk):
    i = pl.program_id(0)

    @pl.when(i == 0)
    def _():
        x = x_ref[...]
        xmean = jnp.mean(jnp.abs(x), axis=0, keepdims=True)
        bits = jax.lax.bitcast_convert_type(xmean, jnp.int32)

        def step(j, t):
            cand = t | jnp.int32(1) << (30 - j)
            cnt = jnp.sum((bits >= cand).astype(jnp.int32))
            return jnp.where(cnt >= topk, cand, t)

        thr = jax.lax.fori_loop(0, 31, step, jnp.int32(0))
        xm_ref[...] = jnp.where(bits >= thr, x, 0.0)

    acc = jax.lax.dot_general(
        xm_ref[...], w_ref[...],
        (((1,), (1,)), ((), ())),
        preferred_element_type=jnp.float32,
    )
    o_ref[...] = acc + b_ref[...]


def kernel(x, weight, bias):
    bsz, seq, in_f = x.shape
    out_f = weight.shape[0]
    topk = int(in_f * 0.1)
    block_r = 256
    assert out_f % block_r == 0

    x2 = x.reshape(bsz * seq, in_f)
    b2 = bias.reshape(1, out_f)

    out = pl.pallas_call(
        functools.partial(_matmul_body, topk=topk),
        grid=(out_f // block_r,),
        in_specs=[
            pl.BlockSpec((bsz * seq, in_f), lambda i: (0, 0)),
            pl.BlockSpec((block_r, in_f), lambda i: (i, 0)),
            pl.BlockSpec((1, block_r), lambda i: (0, i)),
        ],
        out_specs=pl.BlockSpec((bsz * seq, block_r), lambda i: (0, i)),
        out_shape=jax.ShapeDtypeStruct((bsz * seq, out_f), jnp.float32),
        scratch_shapes=[pltpu.VMEM((bsz * seq, in_f), jnp.float32)],
    )(x2, weight, b2)
    return out.reshape(bsz, seq, out_f)


# speedup vs baseline: 2.8285x; 1.1929x over previous
import functools

import jax
import jax.numpy as jnp
from jax.experimental import pallas as pl
from jax.experimental.pallas import tpu as pltpu


def _matmul_body(x_ref, w_ref, b_ref, o_ref, xm_ref, *, topk):
    i = pl.program_id(0)

    @pl.when(i == 0)
    def _():
        x = x_ref[...]
        xmean = jnp.mean(jnp.abs(x), axis=0, keepdims=True)
        bits = jax.lax.bitcast_convert_type(xmean, jnp.int32)

        def step(j, t):
            cand = t | jnp.int32(1) << (30 - j)
            cnt = jnp.sum((bits >= cand).astype(jnp.int32))
            return jnp.where(cnt >= topk, cand, t)

        thr = jax.lax.fori_loop(0, 31, step, jnp.int32(0))
        xm_ref[...] = jnp.where(bits >= thr, x, 0.0)

    acc = jax.lax.dot_general(
        xm_ref[...], w_ref[...],
        (((1,), (1,)), ((), ())),
        preferred_element_type=jnp.float32,
    )
    o_ref[...] = acc + b_ref[...]


def kernel(x, weight, bias):
    bsz, seq, in_f = x.shape
    out_f = weight.shape[0]
    topk = int(in_f * 0.1)
    block_r = 512

    x2 = x.reshape(bsz * seq, in_f)
    b2 = bias.reshape(1, out_f)

    out = pl.pallas_call(
        functools.partial(_matmul_body, topk=topk),
        grid=(pl.cdiv(out_f, block_r),),
        in_specs=[
            pl.BlockSpec((bsz * seq, in_f), lambda i: (0, 0)),
            pl.BlockSpec((block_r, in_f), lambda i: (i, 0)),
            pl.BlockSpec((1, block_r), lambda i: (0, i)),
        ],
        out_specs=pl.BlockSpec((bsz * seq, block_r), lambda i: (0, i)),
        out_shape=jax.ShapeDtypeStruct((bsz * seq, out_f), jnp.float32),
        scratch_shapes=[pltpu.VMEM((bsz * seq, in_f), jnp.float32)],
    )(x2, weight, b2)
    return out.reshape(bsz, seq, out_f)


# speedup vs baseline: 2.8646x; 1.0127x over previous
import functools

import jax
import jax.numpy as jnp
from jax.experimental import pallas as pl
from jax.experimental.pallas import tpu as pltpu


def _matmul_body(x_ref, w_ref, b_ref, o_ref, xm_ref, *, topk):
    i = pl.program_id(0)

    @pl.when(i == 0)
    def _():
        x = x_ref[...]
        xmean = jnp.mean(jnp.abs(x), axis=0, keepdims=True)
        bits = jax.lax.bitcast_convert_type(xmean, jnp.int32)

        def step(j, t):
            cand = t | jnp.int32(1) << (30 - j)
            cnt = jnp.sum((bits >= cand).astype(jnp.int32))
            return jnp.where(cnt >= topk, cand, t)

        thr = jax.lax.fori_loop(0, 31, step, jnp.int32(0))
        xm_ref[...] = jnp.where(bits >= thr, x, 0.0)

    acc = jax.lax.dot_general(
        xm_ref[...], w_ref[...],
        (((1,), (1,)), ((), ())),
        preferred_element_type=jnp.float32,
    )
    o_ref[...] = acc + b_ref[...]


def kernel(x, weight, bias):
    bsz, seq, in_f = x.shape
    out_f = weight.shape[0]
    topk = int(in_f * 0.1)
    block_r = 1024

    x2 = x.reshape(bsz * seq, in_f)
    b2 = bias.reshape(1, out_f)

    out = pl.pallas_call(
        functools.partial(_matmul_body, topk=topk),
        grid=(pl.cdiv(out_f, block_r),),
        in_specs=[
            pl.BlockSpec((bsz * seq, in_f), lambda i: (0, 0)),
            pl.BlockSpec((block_r, in_f), lambda i: (i, 0)),
            pl.BlockSpec((1, block_r), lambda i: (0, i)),
        ],
        out_specs=pl.BlockSpec((bsz * seq, block_r), lambda i: (0, i)),
        out_shape=jax.ShapeDtypeStruct((bsz * seq, out_f), jnp.float32),
        scratch_shapes=[pltpu.VMEM((bsz * seq, in_f), jnp.float32)],
    )(x2, weight, b2)
    return out.reshape(bsz, seq, out_f)


# speedup vs baseline: 2.8664x; 1.0006x over previous
import functools

import jax
import jax.numpy as jnp
from jax.experimental import pallas as pl
from jax.experimental.pallas import tpu as pltpu


def _matmul_body(x_ref, wa_ref, wb_ref, b_ref, o_ref, xm_ref, *, topk):
    i = pl.program_id(0)

    @pl.when(i == 0)
    def _():
        x = x_ref[...]
        xmean = jnp.mean(jnp.abs(x), axis=0, keepdims=True)
        bits = jax.lax.bitcast_convert_type(xmean, jnp.int32)

        def step(j, t):
            cand = t | jnp.int32(1) << (30 - j)
            cnt = jnp.sum((bits >= cand).astype(jnp.int32))
            return jnp.where(cnt >= topk, cand, t)

        thr = jax.lax.fori_loop(0, 31, step, jnp.int32(0))
        xm_ref[...] = jnp.where(bits >= thr, x, 0.0)

    xm = xm_ref[...]
    half = o_ref.shape[1] // 2
    acc_a = jax.lax.dot_general(
        xm, wa_ref[...], (((1,), (1,)), ((), ())),
        preferred_element_type=jnp.float32,
    )
    acc_b = jax.lax.dot_general(
        xm, wb_ref[...], (((1,), (1,)), ((), ())),
        preferred_element_type=jnp.float32,
    )
    b = b_ref[...]
    o_ref[:, :half] = acc_a + b[:, :half]
    o_ref[:, half:] = acc_b + b[:, half:]


def kernel(x, weight, bias):
    bsz, seq, in_f = x.shape
    out_f = weight.shape[0]
    topk = int(in_f * 0.1)
    block_r = 1024

    x2 = x.reshape(bsz * seq, in_f)
    b2 = bias.reshape(1, out_f)

    out = pl.pallas_call(
        functools.partial(_matmul_body, topk=topk),
        grid=(pl.cdiv(out_f, block_r),),
        in_specs=[
            pl.BlockSpec((bsz * seq, in_f), lambda i: (0, 0)),
            pl.BlockSpec((block_r // 2, in_f), lambda i: (2 * i, 0)),
            pl.BlockSpec((block_r // 2, in_f), lambda i: (2 * i + 1, 0)),
            pl.BlockSpec((1, block_r), lambda i: (0, i)),
        ],
        out_specs=pl.BlockSpec((bsz * seq, block_r), lambda i: (0, i)),
        out_shape=jax.ShapeDtypeStruct((bsz * seq, out_f), jnp.float32),
        scratch_shapes=[pltpu.VMEM((bsz * seq, in_f), jnp.float32)],
    )(x2, weight, weight, b2)
    return out.reshape(bsz, seq, out_f)
